# baseline (speedup 1.0000x reference)
import functools

import jax
import jax.numpy as jnp
from jax import lax
from jax.experimental import pallas as pl
from jax.experimental.pallas import tpu as pltpu
from jax.experimental.pallas import tpu_sc as plsc

NUM_CORES = 2
NUM_SUBCORES = 16
NUM_WORKERS = NUM_CORES * NUM_SUBCORES
CHUNK = 128
NBUF = 4


@functools.lru_cache(maxsize=None)
def _make_gather(B, D, dtype_name):
    dtype = jnp.dtype(dtype_name)
    b_per_w = B // NUM_WORKERS
    n_chunks = b_per_w // CHUNK
    n_outer = n_chunks // NBUF
    assert b_per_w * NUM_WORKERS == B
    assert n_chunks * CHUNK == b_per_w
    assert n_outer * NBUF == n_chunks

    mesh = plsc.VectorSubcoreMesh(core_axis_name="c", subcore_axis_name="s")

    @functools.partial(
        pl.kernel,
        mesh=mesh,
        out_type=jax.ShapeDtypeStruct((B, D), dtype),
        scratch_types=(
            [pltpu.VMEM((n_chunks, CHUNK), jnp.int32)]
            + [pltpu.VMEM((CHUNK, D), dtype) for _ in range(NBUF)]
            + [pltpu.SemaphoreType.DMA for _ in range(NBUF)]
        ),
        compiler_params=pltpu.CompilerParams(use_tc_tiling_on_sc=False),
    )
    def gather(table_hbm, idx_hbm, out_hbm, idx_v, *rest):
        bufs = rest[:NBUF]
        sems = rest[NBUF:]
        wid = lax.axis_index("s") * NUM_CORES + lax.axis_index("c")
        base = wid * b_per_w

        pltpu.sync_copy(idx_hbm.at[wid], idx_v)

        def fire(j, b):
            pltpu.async_copy(table_hbm.at[idx_v.at[j]], bufs[b], sems[b])

        def drain(j, b):
            pltpu.make_async_copy(table_hbm.at[idx_v.at[j]], bufs[b],
                                  sems[b]).wait()
            pltpu.sync_copy(bufs[b],
                            out_hbm.at[pl.ds(base + j * CHUNK, CHUNK)])

        for b in range(NBUF):
            fire(b, b)

        def outer(g, carry):
            for b in range(NBUF):
                j = g * NBUF + b
                drain(j, b)
                fire(j + NBUF, b)
            return carry

        lax.fori_loop(0, n_outer - 1, outer, 0)

        for b in range(NBUF):
            drain((n_outer - 1) * NBUF + b, b)

    return gather


def kernel(weight, indices):
    D = weight.shape[1]
    B = 1
    for s in indices.shape:
        B *= s
    b_per_w = B // NUM_WORKERS
    idx3 = indices.astype(jnp.int32).reshape(
        NUM_WORKERS, b_per_w // CHUNK, CHUNK)
    out = _make_gather(B, D, str(weight.dtype))(weight, idx3)
    return out.reshape(indices.shape + (D,))
